# Initial kernel scaffold; baseline (speedup 1.0000x reference)
import jax
import jax.numpy as jnp
from jax.experimental import pallas as pl


def kernel(x, edge_index, W_l0, b_l0, W_r0, W_l1, b_l1, W_r1):
    raise NotImplementedError("write your pallas kernel here")


# speedup vs baseline: 4.7270x; 4.7270x over previous
import functools

import jax
import jax.numpy as jnp
from jax import lax
from jax.experimental import pallas as pl
from jax.experimental.pallas import tpu as pltpu
from jax.experimental.pallas import tpu_sc as plsc

N_NODES = 10000
N_EDGES = 320000
FDIM = 128

_NC = 2
_NS = 16
_NW = _NC * _NS
_EPW = N_EDGES // _NW
_CHUNK = 80
_NCHUNK = _EPW // _CHUNK
_NPAD = 10240
_RPT = _NPAD // _NS

_MESH = plsc.VectorSubcoreMesh(core_axis_name="c", subcore_axis_name="s")


@functools.partial(
    pl.kernel,
    mesh=_MESH,
    out_type=jax.ShapeDtypeStruct((_NC * _NPAD, FDIM), jnp.float32),
    scratch_types=[
        pltpu.VMEM((_CHUNK,), jnp.int32),
        pltpu.VMEM((_CHUNK,), jnp.int32),
        pltpu.VMEM((_CHUNK, FDIM), jnp.float32),
        pltpu.VMEM_SHARED((_NPAD, FDIM), jnp.float32),
        pltpu.SemaphoreType.DMA,
    ],
)
def _edge_agg(p_hbm, src_hbm, dst_hbm, z_hbm, out_hbm,
              idx_s, idx_d, rows, acc, sem):
    cid = lax.axis_index("c")
    sid = lax.axis_index("s")
    wid = sid * _NC + cid

    pltpu.sync_copy(z_hbm, acc.at[pl.ds(sid * _RPT, _RPT)])
    plsc.subcore_barrier()

    def step(i, carry):
        base = pl.multiple_of(wid * _EPW + i * _CHUNK, 8)
        pltpu.sync_copy(src_hbm.at[pl.ds(base, _CHUNK)], idx_s)
        pltpu.sync_copy(dst_hbm.at[pl.ds(base, _CHUNK)], idx_d)
        pltpu.async_copy(p_hbm.at[idx_s], rows, sem).wait()
        pltpu.sync_copy(rows, acc.at[idx_d], add=True)
        return carry

    lax.fori_loop(0, _NCHUNK, step, 0)

    plsc.subcore_barrier()
    row0 = cid * _NPAD + sid * _RPT
    pltpu.sync_copy(acc.at[pl.ds(sid * _RPT, _RPT)],
                    out_hbm.at[pl.ds(row0, _RPT)])


@functools.partial(
    pl.kernel,
    mesh=_MESH,
    out_type=jax.ShapeDtypeStruct((_NC * _NPAD, FDIM), jnp.float32),
    scratch_types=[
        pltpu.VMEM((_CHUNK,), jnp.int32),
        pltpu.VMEM((_CHUNK, FDIM), jnp.float32),
        pltpu.VMEM_SHARED((_NPAD, FDIM), jnp.float32),
    ],
)
def _edge_counts(dst_hbm, z_hbm, ones_hbm, cnt_hbm, idx_d, ones_v, accc):
    cid = lax.axis_index("c")
    sid = lax.axis_index("s")
    wid = sid * _NC + cid

    pltpu.sync_copy(z_hbm, accc.at[pl.ds(sid * _RPT, _RPT)])
    pltpu.sync_copy(ones_hbm, ones_v)
    plsc.subcore_barrier()

    def step(i, carry):
        base = pl.multiple_of(wid * _EPW + i * _CHUNK, 8)
        pltpu.sync_copy(dst_hbm.at[pl.ds(base, _CHUNK)], idx_d)
        pltpu.sync_copy(ones_v, accc.at[idx_d], add=True)
        return carry

    lax.fori_loop(0, _NCHUNK, step, 0)

    plsc.subcore_barrier()
    row0 = cid * _NPAD + sid * _RPT
    pltpu.sync_copy(accc.at[pl.ds(sid * _RPT, _RPT)],
                    cnt_hbm.at[pl.ds(row0, _RPT)])


def _dense_pre(h, wlt, wrt, b2d):
    def body(h_ref, wl_ref, wr_ref, b_ref, p_ref, r_ref):
        hv = h_ref[...]
        p_ref[...] = jnp.dot(hv, wl_ref[...], preferred_element_type=jnp.float32)
        r_ref[...] = jnp.dot(hv, wr_ref[...],
                             preferred_element_type=jnp.float32) + b_ref[...]

    return pl.pallas_call(
        body,
        out_shape=(jax.ShapeDtypeStruct((N_NODES, FDIM), jnp.float32),
                   jax.ShapeDtypeStruct((N_NODES, FDIM), jnp.float32)),
    )(h, wlt, wrt, b2d)


def _combine_relu_dense(parts, cnts, r_prev, wlt, wrt, b2d):
    def body(s_ref, c_ref, r_ref, wl_ref, wr_ref, b_ref, p_ref, rr_ref):
        s = s_ref[0:N_NODES, :] + s_ref[_NPAD:_NPAD + N_NODES, :]
        c = c_ref[0:N_NODES, 0:1] + c_ref[_NPAD:_NPAD + N_NODES, 0:1]
        h = jnp.maximum(s / jnp.maximum(c, 1.0) + r_ref[...], 0.0)
        p_ref[...] = jnp.dot(h, wl_ref[...], preferred_element_type=jnp.float32)
        rr_ref[...] = jnp.dot(h, wr_ref[...],
                              preferred_element_type=jnp.float32) + b_ref[...]

    return pl.pallas_call(
        body,
        out_shape=(jax.ShapeDtypeStruct((N_NODES, FDIM), jnp.float32),
                   jax.ShapeDtypeStruct((N_NODES, FDIM), jnp.float32)),
    )(parts, cnts, r_prev, wlt, wrt, b2d)


def _combine_final(parts, cnts, r_prev):
    def body(s_ref, c_ref, r_ref, o_ref):
        s = s_ref[0:N_NODES, :] + s_ref[_NPAD:_NPAD + N_NODES, :]
        c = c_ref[0:N_NODES, 0:1] + c_ref[_NPAD:_NPAD + N_NODES, 0:1]
        o_ref[...] = s / jnp.maximum(c, 1.0) + r_ref[...]

    return pl.pallas_call(
        body,
        out_shape=jax.ShapeDtypeStruct((N_NODES, FDIM), jnp.float32),
    )(parts, cnts, r_prev)


def kernel(x, edge_index, W_l0, b_l0, W_r0, W_l1, b_l1, W_r1):
    src = edge_index[0]
    dst = edge_index[1]
    z = jnp.zeros((_RPT, FDIM), jnp.float32)
    ones = jnp.zeros((_CHUNK, FDIM), jnp.float32).at[:, 0].set(1.0)

    p0, r0 = _dense_pre(x, W_l0.T, W_r0.T, b_l0.reshape(1, FDIM))
    parts0 = _edge_agg(p0, src, dst, z)
    cnts = _edge_counts(dst, z, ones)
    p1, r1 = _combine_relu_dense(parts0, cnts, r0, W_l1.T, W_r1.T,
                                 b_l1.reshape(1, FDIM))
    parts1 = _edge_agg(p1, src, dst, z)
    return _combine_final(parts1, cnts, r1)


# speedup vs baseline: 7.5490x; 1.5970x over previous
import functools

import jax
import jax.numpy as jnp
from jax import lax
from jax.experimental import pallas as pl
from jax.experimental.pallas import tpu as pltpu
from jax.experimental.pallas import tpu_sc as plsc

N_NODES = 10000
N_EDGES = 320000
FDIM = 128

_NC = 2
_NS = 16
_NW = _NC * _NS
_EPW = N_EDGES // _NW
_CHUNK = 80
_NCHUNK = _EPW // _CHUNK
_NPAD = 10240
_RPT = _NPAD // _NS

_NBUF = 5
_NGRP = _NCHUNK // _NBUF
_ABUF = 4
_AGRP = (_NCHUNK - 1) // _ABUF

_MESH = plsc.VectorSubcoreMesh(core_axis_name="c", subcore_axis_name="s")


@functools.partial(
    pl.kernel,
    mesh=_MESH,
    out_type=jax.ShapeDtypeStruct((_NC * _NPAD, FDIM), jnp.float32),
    scratch_types=[
        pltpu.VMEM((_ABUF, _CHUNK), jnp.int32),
        pltpu.VMEM((_ABUF, _CHUNK), jnp.int32),
        pltpu.VMEM((_ABUF, _CHUNK, FDIM), jnp.float32),
        pltpu.VMEM_SHARED((_NPAD, FDIM), jnp.float32),
    ] + [pltpu.SemaphoreType.DMA] * _ABUF,
)
def _edge_agg(p_hbm, src_hbm, dst_hbm, z_hbm, out_hbm,
              idx_s, idx_d, rows, acc, *sems):
    cid = lax.axis_index("c")
    sid = lax.axis_index("s")
    wid = sid * _NC + cid
    e0 = wid * _EPW

    def fire(b, c):
        base = pl.multiple_of(e0 + c * _CHUNK, 8)
        pltpu.sync_copy(src_hbm.at[pl.ds(base, _CHUNK)], idx_s.at[b])
        pltpu.sync_copy(dst_hbm.at[pl.ds(base, _CHUNK)], idx_d.at[b])
        pltpu.async_copy(p_hbm.at[idx_s.at[b]], rows.at[b], sems[b])

    def drain_scatter(b):
        pltpu.make_async_copy(p_hbm.at[idx_s.at[b]], rows.at[b],
                              sems[b]).wait()
        pltpu.sync_copy(rows.at[b], acc.at[idx_d.at[b]], add=True)

    for b in range(_ABUF):
        fire(b, b)

    pltpu.sync_copy(z_hbm, acc.at[pl.ds(sid * _RPT, _RPT)])
    plsc.subcore_barrier()

    def step(j, carry):
        for b in range(_ABUF):
            drain_scatter(b)

            @pl.when(j < _AGRP - 1)
            def _():
                fire(b, (j + 1) * _ABUF + b)
        return carry

    lax.fori_loop(0, _AGRP, step, 0)

    fire(0, _NCHUNK - 1)
    drain_scatter(0)

    plsc.subcore_barrier()
    row0 = cid * _NPAD + sid * _RPT
    pltpu.sync_copy(acc.at[pl.ds(sid * _RPT, _RPT)],
                    out_hbm.at[pl.ds(row0, _RPT)])


@functools.partial(
    pl.kernel,
    mesh=_MESH,
    out_type=jax.ShapeDtypeStruct((_NC * _NPAD, FDIM), jnp.float32),
    scratch_types=[
        pltpu.VMEM((_NBUF, _CHUNK), jnp.int32),
        pltpu.VMEM((_CHUNK, FDIM), jnp.float32),
        pltpu.VMEM_SHARED((_NPAD, FDIM), jnp.float32),
    ] + [pltpu.SemaphoreType.DMA] * _NBUF,
)
def _edge_counts(dst_hbm, z_hbm, ones_hbm, cnt_hbm, idx_d, ones_v, accc,
                 *sems):
    cid = lax.axis_index("c")
    sid = lax.axis_index("s")
    wid = sid * _NC + cid
    e0 = wid * _EPW

    def fire(b, c):
        base = pl.multiple_of(e0 + c * _CHUNK, 8)
        pltpu.async_copy(dst_hbm.at[pl.ds(base, _CHUNK)], idx_d.at[b],
                         sems[b])

    for b in range(_NBUF):
        fire(b, b)

    pltpu.sync_copy(z_hbm, accc.at[pl.ds(sid * _RPT, _RPT)])
    pltpu.sync_copy(ones_hbm, ones_v)
    plsc.subcore_barrier()

    def step(j, carry):
        for b in range(_NBUF):
            base = pl.multiple_of(e0 + (j * _NBUF + b) * _CHUNK, 8)
            pltpu.make_async_copy(dst_hbm.at[pl.ds(base, _CHUNK)],
                                  idx_d.at[b], sems[b]).wait()
            pltpu.sync_copy(ones_v, accc.at[idx_d.at[b]], add=True)

            @pl.when(j < _NGRP - 1)
            def _():
                fire(b, (j + 1) * _NBUF + b)
        return carry

    lax.fori_loop(0, _NGRP, step, 0)

    plsc.subcore_barrier()
    row0 = cid * _NPAD + sid * _RPT
    pltpu.sync_copy(accc.at[pl.ds(sid * _RPT, _RPT)],
                    cnt_hbm.at[pl.ds(row0, _RPT)])


def _dense_pre(h, wlt, wrt, b2d):
    def body(h_ref, wl_ref, wr_ref, b_ref, p_ref, r_ref):
        hv = h_ref[...]
        p_ref[...] = jnp.dot(hv, wl_ref[...], preferred_element_type=jnp.float32)
        r_ref[...] = jnp.dot(hv, wr_ref[...],
                             preferred_element_type=jnp.float32) + b_ref[...]

    return pl.pallas_call(
        body,
        out_shape=(jax.ShapeDtypeStruct((N_NODES, FDIM), jnp.float32),
                   jax.ShapeDtypeStruct((N_NODES, FDIM), jnp.float32)),
    )(h, wlt, wrt, b2d)


def _combine_relu_dense(parts, cnts, r_prev, wlt, wrt, b2d):
    def body(s_ref, c_ref, r_ref, wl_ref, wr_ref, b_ref, p_ref, rr_ref):
        s = s_ref[0:N_NODES, :] + s_ref[_NPAD:_NPAD + N_NODES, :]
        c = c_ref[0:N_NODES, 0:1] + c_ref[_NPAD:_NPAD + N_NODES, 0:1]
        h = jnp.maximum(s / jnp.maximum(c, 1.0) + r_ref[...], 0.0)
        p_ref[...] = jnp.dot(h, wl_ref[...], preferred_element_type=jnp.float32)
        rr_ref[...] = jnp.dot(h, wr_ref[...],
                              preferred_element_type=jnp.float32) + b_ref[...]

    return pl.pallas_call(
        body,
        out_shape=(jax.ShapeDtypeStruct((N_NODES, FDIM), jnp.float32),
                   jax.ShapeDtypeStruct((N_NODES, FDIM), jnp.float32)),
    )(parts, cnts, r_prev, wlt, wrt, b2d)


def _combine_final(parts, cnts, r_prev):
    def body(s_ref, c_ref, r_ref, o_ref):
        s = s_ref[0:N_NODES, :] + s_ref[_NPAD:_NPAD + N_NODES, :]
        c = c_ref[0:N_NODES, 0:1] + c_ref[_NPAD:_NPAD + N_NODES, 0:1]
        o_ref[...] = s / jnp.maximum(c, 1.0) + r_ref[...]

    return pl.pallas_call(
        body,
        out_shape=jax.ShapeDtypeStruct((N_NODES, FDIM), jnp.float32),
    )(parts, cnts, r_prev)


def kernel(x, edge_index, W_l0, b_l0, W_r0, W_l1, b_l1, W_r1):
    src = edge_index[0]
    dst = edge_index[1]
    z = jnp.zeros((_RPT, FDIM), jnp.float32)
    ones = jnp.zeros((_CHUNK, FDIM), jnp.float32).at[:, 0].set(1.0)

    p0, r0 = _dense_pre(x, W_l0.T, W_r0.T, b_l0.reshape(1, FDIM))
    parts0 = _edge_agg(p0, src, dst, z)
    cnts = _edge_counts(dst, z, ones)
    p1, r1 = _combine_relu_dense(parts0, cnts, r0, W_l1.T, W_r1.T,
                                 b_l1.reshape(1, FDIM))
    parts1 = _edge_agg(p1, src, dst, z)
    return _combine_final(parts1, cnts, r1)


# speedup vs baseline: 11.2498x; 1.4902x over previous
import functools

import jax
import jax.numpy as jnp
from jax import lax
from jax.experimental import pallas as pl
from jax.experimental.pallas import tpu as pltpu
from jax.experimental.pallas import tpu_sc as plsc

N_NODES = 10000
N_EDGES = 320000
FDIM = 128

_NC = 2
_NS = 16
_NW = _NC * _NS
_CHUNK = 80
_NCHUNK = 128
_EPW = _NCHUNK * _CHUNK
_NEPAD = _NW * _EPW
_NPAD = 10240
_RPT = _NPAD // _NS

_ABUF = 3
_AGRP = (_NCHUNK - 2) // _ABUF

_MESH = plsc.VectorSubcoreMesh(core_axis_name="c", subcore_axis_name="s")


@functools.partial(
    pl.kernel,
    mesh=_MESH,
    out_type=jax.ShapeDtypeStruct((_NC * _NPAD, FDIM), jnp.float32),
    scratch_types=[
        pltpu.VMEM((_EPW,), jnp.int32),
        pltpu.VMEM((_ABUF, _CHUNK), jnp.int32),
        pltpu.VMEM((_ABUF, _CHUNK, FDIM), jnp.float32),
        pltpu.VMEM_SHARED((_NPAD, FDIM), jnp.float32),
    ] + [pltpu.SemaphoreType.DMA] * (2 * _ABUF),
)
def _edge_agg(p_hbm, src_hbm, dst_hbm, z_hbm, out_hbm,
              src_v, idx_d, rows, acc, *sems):
    gsem = sems[:_ABUF]
    dsem = sems[_ABUF:]
    cid = lax.axis_index("c")
    sid = lax.axis_index("s")
    wid = sid * _NC + cid
    e0 = wid * _EPW

    pltpu.sync_copy(src_hbm.at[pl.ds(e0, _EPW)], src_v)

    def fire(b, c):
        off = pl.multiple_of(c * _CHUNK, 8)
        base = pl.multiple_of(e0 + c * _CHUNK, 8)
        pltpu.async_copy(dst_hbm.at[pl.ds(base, _CHUNK)], idx_d.at[b],
                         dsem[b])
        pltpu.async_copy(p_hbm.at[src_v.at[pl.ds(off, _CHUNK)]], rows.at[b],
                         gsem[b])

    def drain_scatter(b, c):
        off = pl.multiple_of(c * _CHUNK, 8)
        base = pl.multiple_of(e0 + c * _CHUNK, 8)
        pltpu.make_async_copy(dst_hbm.at[pl.ds(base, _CHUNK)], idx_d.at[b],
                              dsem[b]).wait()
        pltpu.make_async_copy(p_hbm.at[src_v.at[pl.ds(off, _CHUNK)]],
                              rows.at[b], gsem[b]).wait()
        pltpu.sync_copy(rows.at[b], acc.at[idx_d.at[b]], add=True)

    for b in range(_ABUF):
        fire(b, b)

    pltpu.sync_copy(z_hbm, acc.at[pl.ds(sid * _RPT, _RPT)])
    plsc.subcore_barrier()

    def step(j, carry):
        for b in range(_ABUF):
            drain_scatter(b, j * _ABUF + b)

            @pl.when(j < _AGRP - 1)
            def _():
                fire(b, (j + 1) * _ABUF + b)
        return carry

    lax.fori_loop(0, _AGRP, step, 0)

    for t, c in enumerate(range(_AGRP * _ABUF, _NCHUNK)):
        fire(t, c)
    for t, c in enumerate(range(_AGRP * _ABUF, _NCHUNK)):
        drain_scatter(t, c)

    plsc.subcore_barrier()
    row0 = cid * _NPAD + sid * _RPT
    pltpu.sync_copy(acc.at[pl.ds(sid * _RPT, _RPT)],
                    out_hbm.at[pl.ds(row0, _RPT)])


@functools.partial(
    pl.kernel,
    mesh=_MESH,
    out_type=jax.ShapeDtypeStruct((_NC * _NPAD, FDIM), jnp.float32),
    scratch_types=[
        pltpu.VMEM((_NCHUNK, _CHUNK), jnp.int32),
        pltpu.VMEM((_CHUNK, FDIM), jnp.float32),
        pltpu.VMEM_SHARED((_NPAD, FDIM), jnp.float32),
    ],
)
def _edge_counts(dst2d_hbm, z_hbm, ones_hbm, cnt_hbm, dst_v, ones_v, accc):
    cid = lax.axis_index("c")
    sid = lax.axis_index("s")
    wid = sid * _NC + cid

    pltpu.sync_copy(dst2d_hbm.at[pl.ds(wid * _NCHUNK, _NCHUNK)], dst_v)
    pltpu.sync_copy(z_hbm, accc.at[pl.ds(sid * _RPT, _RPT)])
    pltpu.sync_copy(ones_hbm, ones_v)
    plsc.subcore_barrier()

    def step(i, carry):
        pltpu.sync_copy(ones_v, accc.at[dst_v.at[i]], add=True)
        return carry

    lax.fori_loop(0, _NCHUNK, step, 0)

    plsc.subcore_barrier()
    row0 = cid * _NPAD + sid * _RPT
    pltpu.sync_copy(accc.at[pl.ds(sid * _RPT, _RPT)],
                    cnt_hbm.at[pl.ds(row0, _RPT)])


def _dense_pre(h, wlt, wrt, b2d):
    def body(h_ref, wl_ref, wr_ref, b_ref, p_ref, r_ref):
        hv = h_ref[...]
        p_ref[...] = jnp.dot(hv, wl_ref[...], preferred_element_type=jnp.float32)
        r_ref[...] = jnp.dot(hv, wr_ref[...],
                             preferred_element_type=jnp.float32) + b_ref[...]

    return pl.pallas_call(
        body,
        out_shape=(jax.ShapeDtypeStruct((N_NODES, FDIM), jnp.float32),
                   jax.ShapeDtypeStruct((N_NODES, FDIM), jnp.float32)),
    )(h, wlt, wrt, b2d)


def _combine_relu_dense(parts, cnts, r_prev, wlt, wrt, b2d):
    def body(s_ref, c_ref, r_ref, wl_ref, wr_ref, b_ref, p_ref, rr_ref):
        s = s_ref[0:N_NODES, :] + s_ref[_NPAD:_NPAD + N_NODES, :]
        c = c_ref[0:N_NODES, 0:1] + c_ref[_NPAD:_NPAD + N_NODES, 0:1]
        h = jnp.maximum(s / jnp.maximum(c, 1.0) + r_ref[...], 0.0)
        p_ref[...] = jnp.dot(h, wl_ref[...], preferred_element_type=jnp.float32)
        rr_ref[...] = jnp.dot(h, wr_ref[...],
                              preferred_element_type=jnp.float32) + b_ref[...]

    return pl.pallas_call(
        body,
        out_shape=(jax.ShapeDtypeStruct((N_NODES, FDIM), jnp.float32),
                   jax.ShapeDtypeStruct((N_NODES, FDIM), jnp.float32)),
    )(parts, cnts, r_prev, wlt, wrt, b2d)


def _combine_final(parts, cnts, r_prev):
    def body(s_ref, c_ref, r_ref, o_ref):
        s = s_ref[0:N_NODES, :] + s_ref[_NPAD:_NPAD + N_NODES, :]
        c = c_ref[0:N_NODES, 0:1] + c_ref[_NPAD:_NPAD + N_NODES, 0:1]
        o_ref[...] = s / jnp.maximum(c, 1.0) + r_ref[...]

    return pl.pallas_call(
        body,
        out_shape=jax.ShapeDtypeStruct((N_NODES, FDIM), jnp.float32),
    )(parts, cnts, r_prev)


def kernel(x, edge_index, W_l0, b_l0, W_r0, W_l1, b_l1, W_r1):
    n_fill = _NEPAD - N_EDGES
    fill = jnp.arange(n_fill, dtype=jnp.int32)
    src = jnp.concatenate([edge_index[0], fill % N_NODES])
    dst = jnp.concatenate([edge_index[1],
                           N_NODES + fill % (_NPAD - N_NODES)])
    dst2d = dst.reshape(_NW * _NCHUNK, _CHUNK)
    z = jnp.zeros((_RPT, FDIM), jnp.float32)
    ones = jnp.zeros((_CHUNK, FDIM), jnp.float32).at[:, 0].set(1.0)

    p0, r0 = _dense_pre(x, W_l0.T, W_r0.T, b_l0.reshape(1, FDIM))
    parts0 = _edge_agg(p0, src, dst, z)
    cnts = _edge_counts(dst2d, z, ones)
    p1, r1 = _combine_relu_dense(parts0, cnts, r0, W_l1.T, W_r1.T,
                                 b_l1.reshape(1, FDIM))
    parts1 = _edge_agg(p1, src, dst, z)
    return _combine_final(parts1, cnts, r1)
